# baseline (speedup 1.0000x reference)
import functools

import jax
import jax.numpy as jnp
from jax import lax
from jax.experimental import pallas as pl
from jax.experimental.pallas import tpu as pltpu
from jax.experimental.pallas import tpu_sc as plsc

VOCAB = 100000
D = 64
B, H, W = 4, 384, 384
P = H * W
LANES = 16

C = W
CS = C + 1
CG = C // 128
NBUF = 3
OBUF = 2
ISLOT = 6
MSLOT = 2


def _make_kernel(b0):
    info = plsc.get_sparse_core_info()
    NC, NS = info.num_cores, info.num_subcores
    NW = NC * NS
    per_w = P // NW
    rows_w = per_w // C
    assert P % NW == 0 and per_w % C == 0
    unroll = ISLOT
    assert rows_w % unroll == 0

    mesh = plsc.VectorSubcoreMesh(core_axis_name="c", subcore_axis_name="s")

    @functools.partial(
        pl.kernel,
        mesh=mesh,
        compiler_params=pltpu.CompilerParams(
            needs_layout_passes=False,
            use_tc_tiling_on_sc=False,
        ),
        out_type=jax.ShapeDtypeStruct((1, D, H, W), jnp.float32),
        scratch_types=[
            pltpu.VMEM((ISLOT * C,), jnp.int32),
            pltpu.VMEM((MSLOT * C,), jnp.float32),
            [pltpu.VMEM((C, D), jnp.float32)] * NBUF,
            [pltpu.VMEM((D, CS), jnp.float32)] * OBUF,
            [pltpu.SemaphoreType.DMA] * NBUF,
            [pltpu.SemaphoreType.DMA] * OBUF,
            [pltpu.SemaphoreType.DMA] * ISLOT,
            [pltpu.SemaphoreType.DMA] * MSLOT,
        ],
    )
    def k(idx_hbm, mask_hbm, table_hbm, out_hbm, idx_v, mask_v, rows, trans,
          gsem, osem, isem, msem):
        wid = lax.axis_index("s") * NC + lax.axis_index("c")
        r0 = wid * rows_w
        iota = lax.iota(jnp.int32, LANES)
        d_vecs = [kq * LANES + iota for kq in range(D // LANES)]

        def idx_copy(c, sl):
            return pltpu.make_async_copy(
                idx_hbm.at[b0, 0, r0 + c, :],
                idx_v.at[pl.ds(sl * C, C)],
                isem[sl],
            )

        def mask_copy(c, sl):
            return pltpu.make_async_copy(
                mask_hbm.at[b0, 0, r0 + c, :],
                mask_v.at[pl.ds(sl * C, C)],
                msem[sl],
            )

        def gather(sl_i, buf):
            return [
                pltpu.make_async_copy(
                    table_hbm.at[idx_v.at[pl.ds(sl_i * C + j * 128, 128)]],
                    rows[buf].at[pl.ds(j * 128, 128)],
                    gsem[buf],
                )
                for j in range(CG)
            ]

        def out_copy(c, tb):
            return pltpu.make_async_copy(
                trans[tb].at[:, pl.ds(0, C)],
                out_hbm.at[0, :, r0 + c, :],
                osem[tb],
            )

        for sl in range(ISLOT):
            idx_copy(sl, sl).start()
        for sl in range(MSLOT):
            mask_copy(sl, sl).start()
        for c in range(NBUF):
            idx_copy(c, c).wait()
            for cp in gather(c, c):
                cp.start()

        def super_body(s, carry):
            for ks in range(unroll):
                i = s * unroll + ks
                buf = ks % NBUF
                tb = ks % OBUF
                msl = ks % MSLOT
                for cp in gather(ks % ISLOT, buf):
                    cp.wait()
                mask_copy(i, msl).wait()

                @pl.when(i + ISLOT < rows_w)
                def _():
                    idx_copy(i + ISLOT, ks).start()

                @pl.when(i >= OBUF)
                def _():
                    out_copy(i - OBUF, tb).wait()

                def g_body(g, c2, buf=buf, tb=tb, msl=msl):
                    g16 = g * LANES
                    mvec = mask_v[pl.ds(msl * C + g16, LANES)]
                    for p16 in range(LANES):
                        p = g16 + p16
                        mb = jnp.full((LANES,), mvec[p16])
                        p_vec = jnp.full((LANES,), p, jnp.int32)
                        for kq in range(D // LANES):
                            v = rows[buf][p, pl.ds(kq * LANES, LANES)]
                            plsc.store_scatter(
                                trans[tb], [d_vecs[kq], p_vec], v * mb
                            )
                    return c2

                lax.fori_loop(0, C // LANES, g_body, 0)

                @pl.when(i + NBUF < rows_w)
                def _():
                    idx_copy(i + NBUF, (ks + NBUF) % ISLOT).wait()
                    for cp in gather((ks + NBUF) % ISLOT, buf):
                        cp.start()

                @pl.when(i + MSLOT < rows_w)
                def _():
                    mask_copy(i + MSLOT, msl).start()

                out_copy(i, tb).start()
            return carry

        lax.fori_loop(0, rows_w // unroll, super_body, 0)

        for tail in range(OBUF):
            c = rows_w - OBUF + tail
            out_copy(c, c % OBUF).wait()

    return k


_kernels = [_make_kernel(b0) for b0 in range(B)]


@jax.jit
def kernel(vertices, E_mask, embed_map):
    idx = vertices.astype(jnp.int32)
    out = jnp.zeros((B, D, H, W), jnp.float32)
    for b0, k in enumerate(_kernels):
        out = lax.dynamic_update_slice(out, k(idx, E_mask, embed_map),
                                       (b0, 0, 0, 0))
    return out


# speedup vs baseline: 1.1139x; 1.1139x over previous
import functools

import jax
import jax.numpy as jnp
from jax import lax
from jax.experimental import pallas as pl
from jax.experimental.pallas import tpu as pltpu
from jax.experimental.pallas import tpu_sc as plsc

VOCAB = 100000
D = 64
B, H, W = 4, 384, 384
P = H * W
N = B * P
LANES = 16

C = W
CS = C + 1
CG = C // 128
NBUF = 3
OBUF = 2
ISLOT = 6
MSLOT = 2


def _make_kernel():
    info = plsc.get_sparse_core_info()
    NC, NS = info.num_cores, info.num_subcores
    NW = NC * NS
    per_w = N // NW
    rows_w = per_w // C
    assert N % NW == 0 and per_w % C == 0 and P % per_w == 0
    unroll = ISLOT
    assert rows_w % unroll == 0

    mesh = plsc.VectorSubcoreMesh(core_axis_name="c", subcore_axis_name="s")

    @functools.partial(
        pl.kernel,
        mesh=mesh,
        compiler_params=pltpu.CompilerParams(
            needs_layout_passes=False,
            use_tc_tiling_on_sc=False,
        ),
        out_type=jax.ShapeDtypeStruct((B, D, H, W), jnp.float32),
        scratch_types=[
            pltpu.VMEM((ISLOT * C,), jnp.int32),
            pltpu.VMEM((MSLOT * C,), jnp.float32),
            [pltpu.VMEM((C, D), jnp.float32)] * NBUF,
            [pltpu.VMEM((D, CS), jnp.float32)] * OBUF,
            [pltpu.SemaphoreType.DMA] * NBUF,
            [pltpu.SemaphoreType.DMA] * OBUF,
            [pltpu.SemaphoreType.DMA] * ISLOT,
            [pltpu.SemaphoreType.DMA] * MSLOT,
        ],
    )
    def k(idx_hbm, mask_hbm, table_hbm, out_hbm, idx_v, mask_v, rows, trans,
          gsem, osem, isem, msem):
        wid = lax.axis_index("s") * NC + lax.axis_index("c")
        wpi = P // per_w
        b = wid // wpi
        r0 = (wid - b * wpi) * rows_w
        iota = lax.iota(jnp.int32, LANES)
        d_vecs = [kq * LANES + iota for kq in range(D // LANES)]

        def idx_copy(c, sl):
            return pltpu.make_async_copy(
                idx_hbm.at[b, 0, r0 + c, :],
                idx_v.at[pl.ds(sl * C, C)],
                isem[sl],
            )

        def mask_copy(c, sl):
            return pltpu.make_async_copy(
                mask_hbm.at[b, 0, r0 + c, :],
                mask_v.at[pl.ds(sl * C, C)],
                msem[sl],
            )

        def gather(sl_i, buf):
            return [
                pltpu.make_async_copy(
                    table_hbm.at[idx_v.at[pl.ds(sl_i * C + j * 128, 128)]],
                    rows[buf].at[pl.ds(j * 128, 128)],
                    gsem[buf],
                )
                for j in range(CG)
            ]

        def out_copy(c, tb):
            return pltpu.make_async_copy(
                trans[tb].at[:, pl.ds(0, C)],
                out_hbm.at[b, :, r0 + c, :],
                osem[tb],
            )

        for sl in range(ISLOT):
            idx_copy(sl, sl).start()
        for sl in range(MSLOT):
            mask_copy(sl, sl).start()
        for c in range(NBUF):
            idx_copy(c, c).wait()
            for cp in gather(c, c):
                cp.start()

        def super_body(s, carry):
            for ks in range(unroll):
                i = s * unroll + ks
                buf = ks % NBUF
                tb = ks % OBUF
                msl = ks % MSLOT
                for cp in gather(ks % ISLOT, buf):
                    cp.wait()
                mask_copy(i, msl).wait()

                @pl.when(i + ISLOT < rows_w)
                def _():
                    idx_copy(i + ISLOT, ks).start()

                @pl.when(i >= OBUF)
                def _():
                    out_copy(i - OBUF, tb).wait()

                def g_body(g, c2, buf=buf, tb=tb, msl=msl):
                    g16 = g * LANES
                    mvec = mask_v[pl.ds(msl * C + g16, LANES)]
                    for p16 in range(LANES):
                        p = g16 + p16
                        mb = jnp.full((LANES,), mvec[p16])
                        p_vec = jnp.full((LANES,), p, jnp.int32)
                        for kq in range(D // LANES):
                            v = rows[buf][p, pl.ds(kq * LANES, LANES)]
                            plsc.store_scatter(
                                trans[tb], [d_vecs[kq], p_vec], v * mb
                            )
                    return c2

                lax.fori_loop(0, C // LANES, g_body, 0)

                @pl.when(i + NBUF < rows_w)
                def _():
                    idx_copy(i + NBUF, (ks + NBUF) % ISLOT).wait()
                    for cp in gather((ks + NBUF) % ISLOT, buf):
                        cp.start()

                @pl.when(i + MSLOT < rows_w)
                def _():
                    mask_copy(i + MSLOT, msl).start()

                out_copy(i, tb).start()
            return carry

        lax.fori_loop(0, rows_w // unroll, super_body, 0)

        for tail in range(OBUF):
            c = rows_w - OBUF + tail
            out_copy(c, c % OBUF).wait()

    return k


_kernel = _make_kernel()


@jax.jit
def kernel(vertices, E_mask, embed_map):
    return _kernel(vertices.astype(jnp.int32), E_mask, embed_map)
